# baseline (speedup 1.0000x reference)
import functools

import jax
import jax.numpy as jnp
from jax import lax
from jax.experimental import pallas as pl
from jax.experimental.pallas import tpu as pltpu
from jax.experimental.pallas import tpu_sc as plsc

N_NODES = 10000
N_EDGES = 320000
D_IN = 128

NUM_CORES = 2
NUM_SUBCORES = 16
NUM_WORKERS = NUM_CORES * NUM_SUBCORES
CHUNK = 112
EDGES_PER_WORKER = N_EDGES // NUM_WORKERS
N_CHUNKS = EDGES_PER_WORKER // CHUNK
TAIL = EDGES_PER_WORKER - N_CHUNKS * CHUNK
N_PAD = 10240
SEG = N_PAD // NUM_SUBCORES


def _sc_segment_sum(edge_index, x, zeros):
  mesh = plsc.VectorSubcoreMesh(
      core_axis_name="c", subcore_axis_name="s",
      num_cores=NUM_CORES, num_subcores=NUM_SUBCORES)

  @functools.partial(
      pl.kernel,
      out_type=jax.ShapeDtypeStruct((NUM_CORES, N_PAD, D_IN), jnp.float32),
      mesh=mesh,
      scratch_types=[
          pltpu.VMEM((EDGES_PER_WORKER,), jnp.int32),
          pltpu.VMEM((EDGES_PER_WORKER,), jnp.int32),
          pltpu.VMEM((CHUNK, D_IN), jnp.float32),
          pltpu.VMEM((CHUNK, D_IN), jnp.float32),
          pltpu.VMEM_SHARED((N_PAD, D_IN), jnp.float32),
          pltpu.SemaphoreType.DMA,
          pltpu.SemaphoreType.DMA,
      ],
  )
  def sc_kernel(e_hbm, x_hbm, z_hbm, out_hbm,
                src_v, dst_v, rows_a, rows_b, accum, sem_a, sem_b):
    c = lax.axis_index("c")
    s = lax.axis_index("s")
    edge_base = (c * NUM_SUBCORES + s) * EDGES_PER_WORKER

    pltpu.sync_copy(z_hbm, accum.at[pl.ds(s * SEG, SEG)])

    pltpu.sync_copy(e_hbm.at[pl.ds(edge_base, EDGES_PER_WORKER)], src_v)
    pltpu.sync_copy(e_hbm.at[pl.ds(N_EDGES + edge_base, EDGES_PER_WORKER)],
                    dst_v)

    plsc.subcore_barrier()

    def gather_start(off, n, rows_ref, sem):
      pltpu.async_copy(x_hbm.at[src_v.at[pl.ds(off, n)]], rows_ref, sem)

    def gather_wait(off, n, rows_ref, sem):
      pltpu.make_async_copy(x_hbm.at[src_v.at[pl.ds(off, n)]],
                            rows_ref, sem).wait()

    def scatter_add(off, n, rows_ref):
      pltpu.sync_copy(rows_ref, accum.at[dst_v.at[pl.ds(off, n)]], add=True)

    gather_start(0, CHUNK, rows_a, sem_a)

    def body(i, carry):
      gather_start((2 * i + 1) * CHUNK, CHUNK, rows_b, sem_b)
      gather_wait(2 * i * CHUNK, CHUNK, rows_a, sem_a)
      scatter_add(2 * i * CHUNK, CHUNK, rows_a)
      gather_start((2 * i + 2) * CHUNK, CHUNK, rows_a, sem_a)
      gather_wait((2 * i + 1) * CHUNK, CHUNK, rows_b, sem_b)
      scatter_add((2 * i + 1) * CHUNK, CHUNK, rows_b)
      return carry

    lax.fori_loop(0, N_CHUNKS // 2, body, 0)

    gather_start(N_CHUNKS * CHUNK, TAIL, rows_b.at[pl.ds(0, TAIL)], sem_b)
    gather_wait((N_CHUNKS - 1) * CHUNK, CHUNK, rows_a, sem_a)
    scatter_add((N_CHUNKS - 1) * CHUNK, CHUNK, rows_a)
    gather_wait(N_CHUNKS * CHUNK, TAIL, rows_b.at[pl.ds(0, TAIL)], sem_b)
    scatter_add(N_CHUNKS * CHUNK, TAIL, rows_b.at[pl.ds(0, TAIL)])

    plsc.subcore_barrier()

    pltpu.sync_copy(accum.at[pl.ds(s * SEG, SEG)],
                    out_hbm.at[c, pl.ds(s * SEG, SEG)])

  return sc_kernel(edge_index, x, zeros)


BLK = 1000
NBLK = N_NODES // BLK
_DIMS = (((1,), (1,)), ((), ()))


def _mlp_body(p_ref, x_ref, eps_ref, w1_ref, b1_ref, g1_ref, be1_ref,
              w2_ref, b2_ref, g2_ref, be2_ref, o_ref,
              pa0, pa1, xa, pb0, pb1, xb, h_scr, h2_scr, oa, ob,
              sem_a, sem_b, sem_oa, sem_ob):
  eps = eps_ref[0, 0]
  bufs = [(pa0, pa1, xa, sem_a), (pb0, pb1, xb, sem_b)]

  def srcs(kb):
    r = pl.ds(kb * BLK, BLK)
    return (p_ref.at[0, r, :], p_ref.at[1, r, :], x_ref.at[r, :])

  def start(kb, b):
    for s, d in zip(srcs(kb), b[:3]):
      pltpu.async_copy(s, d, b[3])

  def wait(kb, b):
    for s, d in zip(srcs(kb), b[:3]):
      pltpu.make_async_copy(s, d, b[3]).wait()

  start(0, bufs[0])
  s1 = jnp.zeros((1, 256), jnp.float32)
  s2 = jnp.zeros((1, 256), jnp.float32)
  for kb in range(NBLK):
    b = bufs[kb % 2]
    if kb + 1 < NBLK:
      start(kb + 1, bufs[(kb + 1) % 2])
    wait(kb, b)
    vb = b[0][...] + b[1][...] + eps * b[2][...]
    hb = lax.dot_general(vb, w1_ref[...], _DIMS,
                         preferred_element_type=jnp.float32) + b1_ref[...]
    h_scr[pl.ds(kb * BLK, BLK), :] = hb
    s1 = s1 + jnp.sum(hb, axis=0, keepdims=True)
    s2 = s2 + jnp.sum(hb * hb, axis=0, keepdims=True)

  mean1 = s1 * (1.0 / N_NODES)
  var1 = s2 * (1.0 / N_NODES) - mean1 * mean1
  scale1 = lax.rsqrt(var1 + 1e-5) * g1_ref[...]

  t1 = jnp.zeros((1, 128), jnp.float32)
  t2 = jnp.zeros((1, 128), jnp.float32)
  for kb in range(NBLK):
    r = pl.ds(kb * BLK, BLK)
    hn = jnp.maximum((h_scr[r, :] - mean1) * scale1 + be1_ref[...], 0.0)
    h2 = lax.dot_general(hn, w2_ref[...], _DIMS,
                         preferred_element_type=jnp.float32) + b2_ref[...]
    h2_scr[r, :] = h2
    t1 = t1 + jnp.sum(h2, axis=0, keepdims=True)
    t2 = t2 + jnp.sum(h2 * h2, axis=0, keepdims=True)

  mean2 = t1 * (1.0 / N_NODES)
  var2 = t2 * (1.0 / N_NODES) - mean2 * mean2
  scale2 = lax.rsqrt(var2 + 1e-5) * g2_ref[...]

  obufs = [(oa, sem_oa), (ob, sem_ob)]
  for kb in range(NBLK):
    r = pl.ds(kb * BLK, BLK)
    buf, sem = obufs[kb % 2]
    if kb >= 2:
      rp = pl.ds((kb - 2) * BLK, BLK)
      pltpu.make_async_copy(buf, o_ref.at[rp, :], sem).wait()
    buf[...] = jnp.maximum((h2_scr[r, :] - mean2) * scale2 + be2_ref[...],
                           0.0)
    pltpu.async_copy(buf, o_ref.at[r, :], sem)
  for kb in (NBLK - 2, NBLK - 1):
    r = pl.ds(kb * BLK, BLK)
    buf, sem = obufs[kb % 2]
    pltpu.make_async_copy(buf, o_ref.at[r, :], sem).wait()


def _tc_mlp(partials, x, eps, w1, b1, g1, be1, w2, b2, g2, be2):
  hbm = pl.BlockSpec(memory_space=pltpu.MemorySpace.HBM)
  vmem = pl.BlockSpec(memory_space=pltpu.MemorySpace.VMEM)
  return pl.pallas_call(
      _mlp_body,
      in_specs=[hbm, hbm] + [vmem] * 9,
      out_specs=hbm,
      out_shape=jax.ShapeDtypeStruct((N_NODES, w2.shape[0]), jnp.float32),
      scratch_shapes=(
          [pltpu.VMEM((BLK, D_IN), jnp.float32)] * 3 * 2
          + [pltpu.VMEM((N_NODES, 256), jnp.float32),
             pltpu.VMEM((N_NODES, 128), jnp.float32),
             pltpu.VMEM((BLK, 128), jnp.float32),
             pltpu.VMEM((BLK, 128), jnp.float32)]
          + [pltpu.SemaphoreType.DMA] * 4),
  )(partials, x, eps, w1, b1, g1, be1, w2, b2, g2, be2)


@jax.jit
def kernel(x, edge_index, eps, W1, b1, g1, be1, W2, b2, g2, be2):
  zeros = jnp.zeros((SEG, D_IN), jnp.float32)
  partials = _sc_segment_sum(edge_index.reshape(-1), x, zeros)
  return _tc_mlp(partials, x, eps, W1, b1, g1, be1, W2, b2, g2, be2)


# speedup vs baseline: 1.0132x; 1.0132x over previous
import functools

import jax
import jax.numpy as jnp
from jax import lax
from jax.experimental import pallas as pl
from jax.experimental.pallas import tpu as pltpu
from jax.experimental.pallas import tpu_sc as plsc

N_NODES = 10000
N_EDGES = 320000
D_IN = 128

NUM_CORES = 2
NUM_SUBCORES = 16
NUM_WORKERS = NUM_CORES * NUM_SUBCORES
CHUNK = 112
EDGES_PER_WORKER = N_EDGES // NUM_WORKERS
N_CHUNKS = EDGES_PER_WORKER // CHUNK
TAIL = EDGES_PER_WORKER - N_CHUNKS * CHUNK
N_PAD = 10240
SEG = N_PAD // NUM_SUBCORES


def _sc_segment_sum(edge_index, x, zeros):
  mesh = plsc.VectorSubcoreMesh(
      core_axis_name="c", subcore_axis_name="s",
      num_cores=NUM_CORES, num_subcores=NUM_SUBCORES)

  @functools.partial(
      pl.kernel,
      out_type=jax.ShapeDtypeStruct((NUM_CORES, N_PAD, D_IN), jnp.float32),
      mesh=mesh,
      scratch_types=[
          pltpu.VMEM((EDGES_PER_WORKER,), jnp.int32),
          pltpu.VMEM((EDGES_PER_WORKER,), jnp.int32),
          pltpu.VMEM((CHUNK, D_IN), jnp.float32),
          pltpu.VMEM((CHUNK, D_IN), jnp.float32),
          pltpu.VMEM_SHARED((N_PAD, D_IN), jnp.float32),
          pltpu.SemaphoreType.DMA,
          pltpu.SemaphoreType.DMA,
      ],
  )
  def sc_kernel(e_hbm, x_hbm, z_hbm, out_hbm,
                src_v, dst_v, rows_a, rows_b, accum, sem_a, sem_b):
    c = lax.axis_index("c")
    s = lax.axis_index("s")
    edge_base = (c * NUM_SUBCORES + s) * EDGES_PER_WORKER

    pltpu.sync_copy(z_hbm, accum.at[pl.ds(s * SEG, SEG)])

    pltpu.sync_copy(e_hbm.at[pl.ds(edge_base, EDGES_PER_WORKER)], src_v)
    pltpu.sync_copy(e_hbm.at[pl.ds(N_EDGES + edge_base, EDGES_PER_WORKER)],
                    dst_v)

    plsc.subcore_barrier()

    def gather_start(off, n, rows_ref, sem):
      pltpu.async_copy(x_hbm.at[src_v.at[pl.ds(off, n)]], rows_ref, sem)

    def gather_wait(off, n, rows_ref, sem):
      pltpu.make_async_copy(x_hbm.at[src_v.at[pl.ds(off, n)]],
                            rows_ref, sem).wait()

    def scatter_add(off, n, rows_ref):
      pltpu.sync_copy(rows_ref, accum.at[dst_v.at[pl.ds(off, n)]], add=True)

    gather_start(0, CHUNK, rows_a, sem_a)

    def body(i, carry):
      gather_start((2 * i + 1) * CHUNK, CHUNK, rows_b, sem_b)
      gather_wait(2 * i * CHUNK, CHUNK, rows_a, sem_a)
      scatter_add(2 * i * CHUNK, CHUNK, rows_a)
      gather_start((2 * i + 2) * CHUNK, CHUNK, rows_a, sem_a)
      gather_wait((2 * i + 1) * CHUNK, CHUNK, rows_b, sem_b)
      scatter_add((2 * i + 1) * CHUNK, CHUNK, rows_b)
      return carry

    lax.fori_loop(0, N_CHUNKS // 2, body, 0)

    gather_start(N_CHUNKS * CHUNK, TAIL, rows_b.at[pl.ds(0, TAIL)], sem_b)
    gather_wait((N_CHUNKS - 1) * CHUNK, CHUNK, rows_a, sem_a)
    scatter_add((N_CHUNKS - 1) * CHUNK, CHUNK, rows_a)
    gather_wait(N_CHUNKS * CHUNK, TAIL, rows_b.at[pl.ds(0, TAIL)], sem_b)
    scatter_add(N_CHUNKS * CHUNK, TAIL, rows_b.at[pl.ds(0, TAIL)])

    plsc.subcore_barrier()

    pltpu.sync_copy(accum.at[pl.ds(s * SEG, SEG)],
                    out_hbm.at[c, pl.ds(s * SEG, SEG)])

  return sc_kernel(edge_index, x, zeros)


BLK = 1000
NBLK = N_NODES // BLK
_DIMS = (((1,), (1,)), ((), ()))


def _mlp_body(p_ref, x_ref, eps_ref, w1_ref, b1_ref, g1_ref, be1_ref,
              w2_ref, b2_ref, g2_ref, be2_ref, o_ref,
              pa0, pa1, xa, pb0, pb1, xb, h_scr, h2_scr, sem_a, sem_b):
  eps = eps_ref[0, 0]
  bufs = [(pa0, pa1, xa, sem_a), (pb0, pb1, xb, sem_b)]

  def srcs(kb):
    r = pl.ds(kb * BLK, BLK)
    return (p_ref.at[0, r, :], p_ref.at[1, r, :], x_ref.at[r, :])

  def start(kb, b):
    for s, d in zip(srcs(kb), b[:3]):
      pltpu.async_copy(s, d, b[3])

  def wait(kb, b):
    for s, d in zip(srcs(kb), b[:3]):
      pltpu.make_async_copy(s, d, b[3]).wait()

  start(0, bufs[0])
  s1 = jnp.zeros((1, 256), jnp.float32)
  s2 = jnp.zeros((1, 256), jnp.float32)
  for kb in range(NBLK):
    b = bufs[kb % 2]
    if kb + 1 < NBLK:
      start(kb + 1, bufs[(kb + 1) % 2])
    wait(kb, b)
    vb = b[0][...] + b[1][...] + eps * b[2][...]
    hb = lax.dot_general(vb, w1_ref[...], _DIMS,
                         preferred_element_type=jnp.float32) + b1_ref[...]
    h_scr[pl.ds(kb * BLK, BLK), :] = hb
    s1 = s1 + jnp.sum(hb, axis=0, keepdims=True)
    s2 = s2 + jnp.sum(hb * hb, axis=0, keepdims=True)

  mean1 = s1 * (1.0 / N_NODES)
  var1 = s2 * (1.0 / N_NODES) - mean1 * mean1
  scale1 = lax.rsqrt(var1 + 1e-5) * g1_ref[...]

  t1 = jnp.zeros((1, 128), jnp.float32)
  t2 = jnp.zeros((1, 128), jnp.float32)
  for kb in range(NBLK):
    r = pl.ds(kb * BLK, BLK)
    hn = jnp.maximum((h_scr[r, :] - mean1) * scale1 + be1_ref[...], 0.0)
    h2 = lax.dot_general(hn, w2_ref[...], _DIMS,
                         preferred_element_type=jnp.float32) + b2_ref[...]
    h2_scr[r, :] = h2
    t1 = t1 + jnp.sum(h2, axis=0, keepdims=True)
    t2 = t2 + jnp.sum(h2 * h2, axis=0, keepdims=True)

  mean2 = t1 * (1.0 / N_NODES)
  var2 = t2 * (1.0 / N_NODES) - mean2 * mean2
  scale2 = lax.rsqrt(var2 + 1e-5) * g2_ref[...]

  o_ref[...] = jnp.maximum(
      (h2_scr[...] - mean2) * scale2 + be2_ref[...], 0.0)


def _tc_mlp(partials, x, eps, w1, b1, g1, be1, w2, b2, g2, be2):
  hbm = pl.BlockSpec(memory_space=pltpu.MemorySpace.HBM)
  vmem = pl.BlockSpec(memory_space=pltpu.MemorySpace.VMEM)
  return pl.pallas_call(
      _mlp_body,
      in_specs=[hbm, hbm] + [vmem] * 9,
      out_shape=jax.ShapeDtypeStruct((N_NODES, w2.shape[0]), jnp.float32),
      scratch_shapes=(
          [pltpu.VMEM((BLK, D_IN), jnp.float32)] * 3 * 2
          + [pltpu.VMEM((N_NODES, 256), jnp.float32),
             pltpu.VMEM((N_NODES, 128), jnp.float32),
             pltpu.SemaphoreType.DMA, pltpu.SemaphoreType.DMA]),
  )(partials, x, eps, w1, b1, g1, be1, w2, b2, g2, be2)


@jax.jit
def kernel(x, edge_index, eps, W1, b1, g1, be1, W2, b2, g2, be2):
  zeros = jnp.zeros((SEG, D_IN), jnp.float32)
  partials = _sc_segment_sum(edge_index.reshape(-1), x, zeros)
  return _tc_mlp(partials, x, eps, W1, b1, g1, be1, W2, b2, g2, be2)
